# Initial kernel scaffold; baseline (speedup 1.0000x reference)
import jax
import jax.numpy as jnp
from jax.experimental import pallas as pl


def kernel(x, edge_index, edge_attr, batch_ids, W_atom, W_bond, eps, W_edge, W1, b1, W2, b2, bn_g, bn_b, bne_g, bne_b, bnin_g, bnin_b, fcl_W, fcl_b):
    raise NotImplementedError("write your pallas kernel here")


# speedup vs baseline: 3.0245x; 3.0245x over previous
import functools

import jax
import jax.numpy as jnp
from jax import lax
from jax.experimental import pallas as pl
from jax.experimental.pallas import tpu as pltpu, tpu_sc as plsc

N = 10000
E = 160000
H = 256
HH = 128
ED = 16
L = 3
NG = 256
NT = 128
AV = 64
BV = 16
NAF = 9
NBF = 3

NB_N = 10
BN_N = N // NB_N
NB_E = 80
BE = E // NB_E

SC_CORES = 2
SC_TILES = 16
LANES = 16
EPT = E // SC_TILES
CHUNK = 80
NP = 10240
RPT = NP // SC_TILES


def _f32(v):
  return v.astype(jnp.float32)


def _a1_body(x_ref, wa_ref, xe_ref, part_ref):
  acc = jnp.zeros((BN_N, H), jnp.float32)
  iota = lax.broadcasted_iota(jnp.int32, (BN_N, AV), 1)
  for f in range(NAF):
    oh = _f32(x_ref[:, f : f + 1] == iota)
    acc = acc + jnp.dot(oh, wa_ref[f], preferred_element_type=jnp.float32,
                  precision=lax.Precision.HIGHEST)
  xe_ref[:] = acc
  part_ref[0, :, :H] = jnp.sum(acc, axis=0, keepdims=True)
  part_ref[0, :, H:] = jnp.sum(acc * acc, axis=0, keepdims=True)


def _atom_embed(x, W_atom):
  return pl.pallas_call(
      _a1_body,
      grid=(NB_N,),
      in_specs=[
          pl.BlockSpec((BN_N, NAF), lambda i: (i, 0)),
          pl.BlockSpec((NAF, AV, H), lambda i: (0, 0, 0)),
      ],
      out_specs=[
          pl.BlockSpec((BN_N, H), lambda i: (i, 0)),
          pl.BlockSpec((1, 1, 2 * H), lambda i: (i, 0, 0)),
      ],
      out_shape=[
          jax.ShapeDtypeStruct((N, H), jnp.float32),
          jax.ShapeDtypeStruct((NB_N, 1, 2 * H), jnp.float32),
      ],
  )(x, W_atom)


def _bn_from_partials(part, n_rows, d):
  tot = jnp.sum(part, axis=(0, 1))
  m = tot[:d] * (1.0 / n_rows)
  ex2 = tot[d:] * (1.0 / n_rows)
  var = ex2 - m * m
  inv = lax.rsqrt(var + 1e-5)
  return m, inv


def _a2_body(xe_ref, part_ref, g_ref, b_ref, h0_ref, h1_ref):
  m, inv = _bn_from_partials(part_ref[:], N, H)
  y = g_ref[:] * (xe_ref[:] - m[None, :]) * inv[None, :] + b_ref[:]
  h0_ref[:] = y[:, :HH]
  h1_ref[:] = y[:, HH:]


def _input_bn(xe, part, g, b):
  return pl.pallas_call(
      _a2_body,
      grid=(NB_N,),
      in_specs=[
          pl.BlockSpec((BN_N, H), lambda i: (i, 0)),
          pl.BlockSpec((NB_N, 1, 2 * H), lambda i: (0, 0, 0)),
          pl.BlockSpec((1, H), lambda i: (0, 0)),
          pl.BlockSpec((1, H), lambda i: (0, 0)),
      ],
      out_specs=[
          pl.BlockSpec((BN_N, HH), lambda i: (i, 0)),
          pl.BlockSpec((BN_N, HH), lambda i: (i, 0)),
      ],
      out_shape=[
          jax.ShapeDtypeStruct((N, HH), jnp.float32),
          jax.ShapeDtypeStruct((N, HH), jnp.float32),
      ],
  )(xe, part, g, b)


def _e1_body(ea_ref, wb_ref, e_ref, part_ref):
  acc = jnp.zeros((BE, ED), jnp.float32)
  iota = lax.broadcasted_iota(jnp.int32, (BE, BV), 1)
  for f in range(NBF):
    oh = _f32(ea_ref[:, f : f + 1] == iota)
    acc = acc + jnp.dot(oh, wb_ref[f], preferred_element_type=jnp.float32,
                  precision=lax.Precision.HIGHEST)
  e_ref[:] = acc
  part_ref[0, :, :ED] = jnp.sum(acc, axis=0, keepdims=True)
  part_ref[0, :, ED:] = jnp.sum(acc * acc, axis=0, keepdims=True)


def _bond_embed(ea, Wb):
  return pl.pallas_call(
      _e1_body,
      grid=(NB_E,),
      in_specs=[
          pl.BlockSpec((BE, NBF), lambda i: (i, 0)),
          pl.BlockSpec((NBF, BV, ED), lambda i: (0, 0, 0)),
      ],
      out_specs=[
          pl.BlockSpec((BE, ED), lambda i: (i, 0)),
          pl.BlockSpec((1, 1, 2 * ED), lambda i: (i, 0, 0)),
      ],
      out_shape=[
          jax.ShapeDtypeStruct((E, ED), jnp.float32),
          jax.ShapeDtypeStruct((NB_E, 1, 2 * ED), jnp.float32),
      ],
  )(ea, Wb)


def _e3_body(e_ref, part_ref, g_ref, b_ref, we_ref, ef0_ref, ef1_ref):
  m, inv = _bn_from_partials(part_ref[:], E, ED)
  en = g_ref[:] * (e_ref[:] - m[None, :]) * inv[None, :] + b_ref[:]
  ef = jnp.dot(en.astype(jnp.bfloat16), we_ref[:].astype(jnp.bfloat16),
               preferred_element_type=jnp.float32)
  ef0_ref[:] = ef[:, :HH]
  ef1_ref[:] = ef[:, HH:]


def _edge_project(e, part, g, b, We):
  return pl.pallas_call(
      _e3_body,
      grid=(NB_E,),
      in_specs=[
          pl.BlockSpec((BE, ED), lambda i: (i, 0)),
          pl.BlockSpec((NB_E, 1, 2 * ED), lambda i: (0, 0, 0)),
          pl.BlockSpec((1, ED), lambda i: (0, 0)),
          pl.BlockSpec((1, ED), lambda i: (0, 0)),
          pl.BlockSpec((ED, H), lambda i: (0, 0)),
      ],
      out_specs=[
          pl.BlockSpec((BE, HH), lambda i: (i, 0)),
          pl.BlockSpec((BE, HH), lambda i: (i, 0)),
      ],
      out_shape=[
          jax.ShapeDtypeStruct((E, HH), jnp.float32),
          jax.ShapeDtypeStruct((E, HH), jnp.float32),
      ],
  )(e, part, g, b, We)


def _sc_body(src_r, dst_r, h0_r, h1_r, ef0_r, ef1_r, z_r, out0_r, out1_r,
             sidx, didx, hrows, efrows, acc, sem):
  c = lax.axis_index("c")
  s = lax.axis_index("s")

  def run(h_r, ef_r, out_r):
    row0 = s * RPT
    pltpu.sync_copy(z_r.at[pl.ds(row0, RPT)], acc.at[pl.ds(row0, RPT)])
    plsc.subcore_barrier()

    def chunk(k, carry):
      off = s * EPT + k * CHUNK
      pltpu.sync_copy(src_r.at[pl.ds(off, CHUNK)], sidx)
      pltpu.sync_copy(dst_r.at[pl.ds(off, CHUNK)], didx)
      pltpu.async_copy(h_r.at[sidx], hrows, sem).wait()
      pltpu.sync_copy(ef_r.at[pl.ds(off, CHUNK)], efrows)

      def rowfn(j, carry2):
        for q in range(HH // LANES):
          sl = pl.ds(q * LANES, LANES)
          hrows[j, sl] = jnp.maximum(hrows[j, sl] + efrows[j, sl], 0.0)
        return carry2

      lax.fori_loop(0, CHUNK, rowfn, 0)
      pltpu.sync_copy(hrows, acc.at[didx], add=True)
      return carry

    lax.fori_loop(0, EPT // CHUNK, chunk, 0)
    plsc.subcore_barrier()
    pltpu.sync_copy(acc.at[pl.ds(row0, RPT)], out_r.at[pl.ds(row0, RPT)])

  @pl.when(c == 0)
  def _():
    run(h0_r, ef0_r, out0_r)

  @pl.when(c == 1)
  def _():
    run(h1_r, ef1_r, out1_r)


def _sc_agg(src, dst, h0, h1, ef0, ef1, zrows):
  sc_call = functools.partial(
      pl.kernel,
      out_type=[
          jax.ShapeDtypeStruct((NP, HH), jnp.float32),
          jax.ShapeDtypeStruct((NP, HH), jnp.float32),
      ],
      mesh=plsc.VectorSubcoreMesh(core_axis_name="c", subcore_axis_name="s"),
      scratch_types=[
          pltpu.VMEM((CHUNK,), jnp.int32),
          pltpu.VMEM((CHUNK,), jnp.int32),
          pltpu.VMEM((CHUNK, HH), jnp.float32),
          pltpu.VMEM((CHUNK, HH), jnp.float32),
          pltpu.VMEM_SHARED((NP, HH), jnp.float32),
          pltpu.SemaphoreType.DMA,
      ],
  )
  return sc_call(_sc_body)(src, dst, h0, h1, ef0, ef1, zrows)


def _m1_body(h0_ref, h1_ref, a0_ref, a1_ref, ep_ref, w1_ref, b1_ref,
             w2_ref, b2_ref, u_ref, part_ref):
  ep = ep_ref[0, 0]
  z0 = (h0_ref[:] * ep + a0_ref[:]).astype(jnp.bfloat16)
  z1 = (h1_ref[:] * ep + a1_ref[:]).astype(jnp.bfloat16)
  w1b = w1_ref[:].astype(jnp.bfloat16)
  t = jnp.dot(z0, w1b[:HH, :], preferred_element_type=jnp.float32)
  t = t + jnp.dot(z1, w1b[HH:, :], preferred_element_type=jnp.float32)
  t = jnp.maximum(t + b1_ref[:], 0.0)
  u = jnp.dot(t.astype(jnp.bfloat16), w2_ref[:].astype(jnp.bfloat16),
              preferred_element_type=jnp.float32) + b2_ref[:]
  u = jnp.maximum(u, 0.0)
  u_ref[:] = u
  part_ref[0, :, :H] = jnp.sum(u, axis=0, keepdims=True)
  part_ref[0, :, H:] = jnp.sum(u * u, axis=0, keepdims=True)


def _mlp(h0, h1, a0, a1, epsp, W1, b1, W2, b2):
  return pl.pallas_call(
      _m1_body,
      grid=(NB_N,),
      in_specs=[
          pl.BlockSpec((BN_N, HH), lambda i: (i, 0)),
          pl.BlockSpec((BN_N, HH), lambda i: (i, 0)),
          pl.BlockSpec((BN_N, HH), lambda i: (i, 0)),
          pl.BlockSpec((BN_N, HH), lambda i: (i, 0)),
          pl.BlockSpec((1, 1), lambda i: (0, 0)),
          pl.BlockSpec((H, H), lambda i: (0, 0)),
          pl.BlockSpec((1, H), lambda i: (0, 0)),
          pl.BlockSpec((H, H), lambda i: (0, 0)),
          pl.BlockSpec((1, H), lambda i: (0, 0)),
      ],
      out_specs=[
          pl.BlockSpec((BN_N, H), lambda i: (i, 0)),
          pl.BlockSpec((1, 1, 2 * H), lambda i: (i, 0, 0)),
      ],
      out_shape=[
          jax.ShapeDtypeStruct((N, H), jnp.float32),
          jax.ShapeDtypeStruct((NB_N, 1, 2 * H), jnp.float32),
      ],
  )(h0, h1, a0, a1, epsp, W1, b1, W2, b2)


def _m2_body(u_ref, part_ref, g_ref, b_ref, h0_ref, h1_ref, n0_ref, n1_ref):
  m, inv = _bn_from_partials(part_ref[:], N, H)
  y = g_ref[:] * (u_ref[:] - m[None, :]) * inv[None, :] + b_ref[:]
  n0_ref[:] = y[:, :HH] + h0_ref[:]
  n1_ref[:] = y[:, HH:] + h1_ref[:]


def _bn_residual(u, part, g, b, h0, h1):
  return pl.pallas_call(
      _m2_body,
      grid=(NB_N,),
      in_specs=[
          pl.BlockSpec((BN_N, H), lambda i: (i, 0)),
          pl.BlockSpec((NB_N, 1, 2 * H), lambda i: (0, 0, 0)),
          pl.BlockSpec((1, H), lambda i: (0, 0)),
          pl.BlockSpec((1, H), lambda i: (0, 0)),
          pl.BlockSpec((BN_N, HH), lambda i: (i, 0)),
          pl.BlockSpec((BN_N, HH), lambda i: (i, 0)),
      ],
      out_specs=[
          pl.BlockSpec((BN_N, HH), lambda i: (i, 0)),
          pl.BlockSpec((BN_N, HH), lambda i: (i, 0)),
      ],
      out_shape=[
          jax.ShapeDtypeStruct((N, HH), jnp.float32),
          jax.ShapeDtypeStruct((N, HH), jnp.float32),
      ],
  )(u, part, g, b, h0, h1)


def _r_body(h0_ref, h1_ref, bid_ref, w_ref, b_ref, out_ref, pool_acc, cnt_acc):
  i = pl.program_id(0)

  @pl.when(i == 0)
  def _():
    pool_acc[:] = jnp.zeros_like(pool_acc)
    cnt_acc[:] = jnp.zeros_like(cnt_acc)

  iota = lax.broadcasted_iota(jnp.int32, (BN_N, NG), 1)
  a = _f32(bid_ref[:] == iota)
  hfull = jnp.concatenate([h0_ref[:], h1_ref[:]], axis=1)
  dn = (((0,), (0,)), ((), ()))
  pool_acc[:] = pool_acc[:] + lax.dot_general(
      a, hfull, dn, preferred_element_type=jnp.float32,
                  precision=lax.Precision.HIGHEST)
  cnt_acc[:] = cnt_acc[:] + lax.dot_general(
      a, jnp.ones((BN_N, NT), jnp.float32), dn,
      preferred_element_type=jnp.float32,
                  precision=lax.Precision.HIGHEST)

  @pl.when(i == NB_N - 1)
  def _():
    cnt = jnp.maximum(cnt_acc[:, 0:1], 1.0)
    pooled = pool_acc[:] / cnt
    out_ref[:] = jnp.dot(
        pooled.astype(jnp.bfloat16), w_ref[:].astype(jnp.bfloat16),
        preferred_element_type=jnp.float32) + b_ref[:]


def _readout(h0, h1, bid, fcl_W, fcl_b):
  return pl.pallas_call(
      _r_body,
      grid=(NB_N,),
      in_specs=[
          pl.BlockSpec((BN_N, HH), lambda i: (i, 0)),
          pl.BlockSpec((BN_N, HH), lambda i: (i, 0)),
          pl.BlockSpec((BN_N, 1), lambda i: (i, 0)),
          pl.BlockSpec((H, NT), lambda i: (0, 0)),
          pl.BlockSpec((1, NT), lambda i: (0, 0)),
      ],
      out_specs=pl.BlockSpec((NG, NT), lambda i: (0, 0)),
      out_shape=jax.ShapeDtypeStruct((NG, NT), jnp.float32),
      scratch_shapes=[
          pltpu.VMEM((NG, H), jnp.float32),
          pltpu.VMEM((NG, NT), jnp.float32),
      ],
  )(h0, h1, bid, fcl_W, fcl_b)


@jax.jit
def kernel(x, edge_index, edge_attr, batch_ids, W_atom, W_bond, eps, W_edge,
           W1, b1, W2, b2, bn_g, bn_b, bne_g, bne_b, bnin_g, bnin_b,
           fcl_W, fcl_b):
  x = x.astype(jnp.int32)
  ea = edge_attr.astype(jnp.int32)
  src = edge_index[0].astype(jnp.int32)
  dst = edge_index[1].astype(jnp.int32)
  bid = batch_ids.astype(jnp.int32).reshape(-1, 1)

  xe, pa = _atom_embed(x, W_atom)
  h0, h1 = _input_bn(xe, pa, bnin_g.reshape(1, -1), bnin_b.reshape(1, -1))
  zrows = jnp.zeros((NP, HH), jnp.float32)

  for i in range(L):
    e, pe = _bond_embed(ea, W_bond[i])
    ef0, ef1 = _edge_project(e, pe, bne_g[i].reshape(1, -1),
                             bne_b[i].reshape(1, -1), W_edge[i])
    a0, a1 = _sc_agg(src, dst, h0, h1, ef0, ef1, zrows)
    a0 = a0[:N]
    a1 = a1[:N]
    epsp = (1.0 + eps[i]).reshape(1, 1)
    u, pu = _mlp(h0, h1, a0, a1, epsp, W1[i], b1[i].reshape(1, -1),
                 W2[i], b2[i].reshape(1, -1))
    h0, h1 = _bn_residual(u, pu, bn_g[i].reshape(1, -1),
                          bn_b[i].reshape(1, -1), h0, h1)

  return _readout(h0, h1, bid, fcl_W, fcl_b.reshape(1, -1))


# speedup vs baseline: 4.0178x; 1.3284x over previous
import functools

import jax
import jax.numpy as jnp
from jax import lax
from jax.experimental import pallas as pl
from jax.experimental.pallas import tpu as pltpu, tpu_sc as plsc

N = 10000
E = 160000
H = 256
HH = 128
ED = 16
L = 3
NG = 256
NT = 128
AV = 64
BV = 16
NAF = 9
NBF = 3

NB_N = 10
BN_N = N // NB_N
NB_E = 80
BE = E // NB_E

SC_CORES = 2
SC_TILES = 16
LANES = 16
EPT = E // SC_TILES
CHUNK = 40
NP = 10240
RPT = NP // SC_TILES


def _f32(v):
  return v.astype(jnp.float32)


def _a1_body(x_ref, wa_ref, xe_ref, part_ref):
  acc = jnp.zeros((BN_N, H), jnp.float32)
  iota = lax.broadcasted_iota(jnp.int32, (BN_N, AV), 1)
  for f in range(NAF):
    oh = _f32(x_ref[:, f : f + 1] == iota)
    acc = acc + jnp.dot(oh, wa_ref[f], preferred_element_type=jnp.float32,
                  precision=lax.Precision.HIGHEST)
  xe_ref[:] = acc
  part_ref[0, :, :H] = jnp.sum(acc, axis=0, keepdims=True)
  part_ref[0, :, H:] = jnp.sum(acc * acc, axis=0, keepdims=True)


def _atom_embed(x, W_atom):
  return pl.pallas_call(
      _a1_body,
      grid=(NB_N,),
      in_specs=[
          pl.BlockSpec((BN_N, NAF), lambda i: (i, 0)),
          pl.BlockSpec((NAF, AV, H), lambda i: (0, 0, 0)),
      ],
      out_specs=[
          pl.BlockSpec((BN_N, H), lambda i: (i, 0)),
          pl.BlockSpec((1, 1, 2 * H), lambda i: (i, 0, 0)),
      ],
      out_shape=[
          jax.ShapeDtypeStruct((N, H), jnp.float32),
          jax.ShapeDtypeStruct((NB_N, 1, 2 * H), jnp.float32),
      ],
  )(x, W_atom)


def _bn_from_partials(part, n_rows, d):
  tot = jnp.sum(part, axis=(0, 1))
  m = tot[:d] * (1.0 / n_rows)
  ex2 = tot[d:] * (1.0 / n_rows)
  var = ex2 - m * m
  inv = lax.rsqrt(var + 1e-5)
  return m, inv


def _a2_body(xe_ref, part_ref, g_ref, b_ref, h0_ref, h1_ref):
  m, inv = _bn_from_partials(part_ref[:], N, H)
  y = g_ref[:] * (xe_ref[:] - m[None, :]) * inv[None, :] + b_ref[:]
  h0_ref[:] = y[:, :HH]
  h1_ref[:] = y[:, HH:]


def _input_bn(xe, part, g, b):
  return pl.pallas_call(
      _a2_body,
      grid=(NB_N,),
      in_specs=[
          pl.BlockSpec((BN_N, H), lambda i: (i, 0)),
          pl.BlockSpec((NB_N, 1, 2 * H), lambda i: (0, 0, 0)),
          pl.BlockSpec((1, H), lambda i: (0, 0)),
          pl.BlockSpec((1, H), lambda i: (0, 0)),
      ],
      out_specs=[
          pl.BlockSpec((BN_N, HH), lambda i: (i, 0)),
          pl.BlockSpec((BN_N, HH), lambda i: (i, 0)),
      ],
      out_shape=[
          jax.ShapeDtypeStruct((N, HH), jnp.float32),
          jax.ShapeDtypeStruct((N, HH), jnp.float32),
      ],
  )(xe, part, g, b)


def _e1_body(ea_ref, wb_ref, e_ref, part_ref):
  acc = jnp.zeros((BE, ED), jnp.float32)
  iota = lax.broadcasted_iota(jnp.int32, (BE, BV), 1)
  for f in range(NBF):
    oh = _f32(ea_ref[:, f : f + 1] == iota)
    acc = acc + jnp.dot(oh, wb_ref[f], preferred_element_type=jnp.float32,
                  precision=lax.Precision.HIGHEST)
  e_ref[:] = acc
  part_ref[0, :, :ED] = jnp.sum(acc, axis=0, keepdims=True)
  part_ref[0, :, ED:] = jnp.sum(acc * acc, axis=0, keepdims=True)


def _bond_embed(ea, Wb):
  return pl.pallas_call(
      _e1_body,
      grid=(NB_E,),
      in_specs=[
          pl.BlockSpec((BE, NBF), lambda i: (i, 0)),
          pl.BlockSpec((NBF, BV, ED), lambda i: (0, 0, 0)),
      ],
      out_specs=[
          pl.BlockSpec((BE, ED), lambda i: (i, 0)),
          pl.BlockSpec((1, 1, 2 * ED), lambda i: (i, 0, 0)),
      ],
      out_shape=[
          jax.ShapeDtypeStruct((E, ED), jnp.float32),
          jax.ShapeDtypeStruct((NB_E, 1, 2 * ED), jnp.float32),
      ],
  )(ea, Wb)


def _e3_body(e_ref, part_ref, g_ref, b_ref, we_ref, ef0_ref, ef1_ref):
  m, inv = _bn_from_partials(part_ref[:], E, ED)
  en = g_ref[:] * (e_ref[:] - m[None, :]) * inv[None, :] + b_ref[:]
  ef = jnp.dot(en.astype(jnp.bfloat16), we_ref[:].astype(jnp.bfloat16),
               preferred_element_type=jnp.float32)
  ef0_ref[:] = ef[:, :HH]
  ef1_ref[:] = ef[:, HH:]


def _edge_project(e, part, g, b, We):
  return pl.pallas_call(
      _e3_body,
      grid=(NB_E,),
      in_specs=[
          pl.BlockSpec((BE, ED), lambda i: (i, 0)),
          pl.BlockSpec((NB_E, 1, 2 * ED), lambda i: (0, 0, 0)),
          pl.BlockSpec((1, ED), lambda i: (0, 0)),
          pl.BlockSpec((1, ED), lambda i: (0, 0)),
          pl.BlockSpec((ED, H), lambda i: (0, 0)),
      ],
      out_specs=[
          pl.BlockSpec((BE, HH), lambda i: (i, 0)),
          pl.BlockSpec((BE, HH), lambda i: (i, 0)),
      ],
      out_shape=[
          jax.ShapeDtypeStruct((E, HH), jnp.float32),
          jax.ShapeDtypeStruct((E, HH), jnp.float32),
      ],
  )(e, part, g, b, We)


NBUF = 4
NCH = EPT // CHUNK
MAIN = (NCH // NBUF) * NBUF
PEEL = NCH - MAIN


def _sc_body(src_r, dst_r, h0_r, h1_r, ef0_r, ef1_r, z_r, out0_r, out1_r,
             *scr):
  sidx = list(scr[0:NBUF])
  didx = list(scr[NBUF:2 * NBUF])
  hbuf = list(scr[2 * NBUF:3 * NBUF])
  efbuf = list(scr[3 * NBUF:4 * NBUF])
  acc = scr[4 * NBUF]
  gsem = list(scr[4 * NBUF + 1:5 * NBUF + 1])
  ssem = list(scr[5 * NBUF + 1:6 * NBUF + 1])
  c = lax.axis_index("c")
  s = lax.axis_index("s")

  def run(h_r, ef_r, out_r):
    row0 = s * RPT
    pltpu.sync_copy(z_r.at[pl.ds(row0, RPT)], acc.at[pl.ds(row0, RPT)])
    plsc.subcore_barrier()
    ebase = s * EPT

    def prep(k, b):
      off = ebase + k * CHUNK
      pltpu.sync_copy(src_r.at[pl.ds(off, CHUNK)], sidx[b])
      pltpu.sync_copy(dst_r.at[pl.ds(off, CHUNK)], didx[b])
      pltpu.async_copy(h_r.at[sidx[b]], hbuf[b], gsem[b])
      pltpu.async_copy(ef_r.at[pl.ds(off, CHUNK)], efbuf[b], gsem[b])

    def consume(k, b):
      off = ebase + k * CHUNK
      pltpu.make_async_copy(h_r.at[sidx[b]], hbuf[b], gsem[b]).wait()
      pltpu.make_async_copy(ef_r.at[pl.ds(off, CHUNK)], efbuf[b],
                            gsem[b]).wait()

      @plsc.parallel_loop(0, CHUNK, 1, unroll=4)
      def rowfn(j):
        for q in range(HH // LANES):
          sl = pl.ds(q * LANES, LANES)
          hbuf[b][j, sl] = jnp.maximum(hbuf[b][j, sl] + efbuf[b][j, sl],
                                       0.0)

      pltpu.async_copy(hbuf[b], acc.at[didx[b]], ssem[b], add=True)

    prep(0, 0)

    @pl.loop(0, MAIN, step=NBUF)
    def chunkgrp(k0):
      for db in range(NBUF):
        k = k0 + db
        bn = (db + 1) % NBUF

        @pl.when(k + 1 < NCH)
        def _():
          @pl.when(k >= NBUF - 1)
          def _():
            pltpu.make_async_copy(hbuf[bn], acc.at[didx[bn]], ssem[bn]).wait()
          prep(k + 1, bn)

        consume(k, db)

    for j in range(1, PEEL):
      pltpu.make_async_copy(hbuf[j], acc.at[didx[j]], ssem[j]).wait()
      prep(MAIN + j, j)
    for j in range(PEEL):
      consume(MAIN + j, j)

    for b in range(NBUF):
      pltpu.make_async_copy(hbuf[b], acc.at[didx[b]], ssem[b]).wait()
    plsc.subcore_barrier()
    pltpu.sync_copy(acc.at[pl.ds(row0, RPT)], out_r.at[pl.ds(row0, RPT)])

  @pl.when(c == 0)
  def _():
    run(h0_r, ef0_r, out0_r)

  @pl.when(c == 1)
  def _():
    run(h1_r, ef1_r, out1_r)


def _sc_agg(src, dst, h0, h1, ef0, ef1, zrows):
  sc_call = functools.partial(
      pl.kernel,
      out_type=[
          jax.ShapeDtypeStruct((NP, HH), jnp.float32),
          jax.ShapeDtypeStruct((NP, HH), jnp.float32),
      ],
      mesh=plsc.VectorSubcoreMesh(core_axis_name="c", subcore_axis_name="s"),
      scratch_types=(
          [pltpu.VMEM((CHUNK,), jnp.int32) for _ in range(2 * NBUF)]
          + [pltpu.VMEM((CHUNK, HH), jnp.float32) for _ in range(2 * NBUF)]
          + [pltpu.VMEM_SHARED((NP, HH), jnp.float32)]
          + [pltpu.SemaphoreType.DMA for _ in range(2 * NBUF)]
      ),
  )
  return sc_call(_sc_body)(src, dst, h0, h1, ef0, ef1, zrows)


def _m1_body(h0_ref, h1_ref, a0_ref, a1_ref, ep_ref, w1_ref, b1_ref,
             w2_ref, b2_ref, u_ref, part_ref):
  ep = ep_ref[0, 0]
  z0 = (h0_ref[:] * ep + a0_ref[:]).astype(jnp.bfloat16)
  z1 = (h1_ref[:] * ep + a1_ref[:]).astype(jnp.bfloat16)
  w1b = w1_ref[:].astype(jnp.bfloat16)
  t = jnp.dot(z0, w1b[:HH, :], preferred_element_type=jnp.float32)
  t = t + jnp.dot(z1, w1b[HH:, :], preferred_element_type=jnp.float32)
  t = jnp.maximum(t + b1_ref[:], 0.0)
  u = jnp.dot(t.astype(jnp.bfloat16), w2_ref[:].astype(jnp.bfloat16),
              preferred_element_type=jnp.float32) + b2_ref[:]
  u = jnp.maximum(u, 0.0)
  u_ref[:] = u
  part_ref[0, :, :H] = jnp.sum(u, axis=0, keepdims=True)
  part_ref[0, :, H:] = jnp.sum(u * u, axis=0, keepdims=True)


def _mlp(h0, h1, a0, a1, epsp, W1, b1, W2, b2):
  return pl.pallas_call(
      _m1_body,
      grid=(NB_N,),
      in_specs=[
          pl.BlockSpec((BN_N, HH), lambda i: (i, 0)),
          pl.BlockSpec((BN_N, HH), lambda i: (i, 0)),
          pl.BlockSpec((BN_N, HH), lambda i: (i, 0)),
          pl.BlockSpec((BN_N, HH), lambda i: (i, 0)),
          pl.BlockSpec((1, 1), lambda i: (0, 0)),
          pl.BlockSpec((H, H), lambda i: (0, 0)),
          pl.BlockSpec((1, H), lambda i: (0, 0)),
          pl.BlockSpec((H, H), lambda i: (0, 0)),
          pl.BlockSpec((1, H), lambda i: (0, 0)),
      ],
      out_specs=[
          pl.BlockSpec((BN_N, H), lambda i: (i, 0)),
          pl.BlockSpec((1, 1, 2 * H), lambda i: (i, 0, 0)),
      ],
      out_shape=[
          jax.ShapeDtypeStruct((N, H), jnp.float32),
          jax.ShapeDtypeStruct((NB_N, 1, 2 * H), jnp.float32),
      ],
  )(h0, h1, a0, a1, epsp, W1, b1, W2, b2)


def _m2_body(u_ref, part_ref, g_ref, b_ref, h0_ref, h1_ref, n0_ref, n1_ref):
  m, inv = _bn_from_partials(part_ref[:], N, H)
  y = g_ref[:] * (u_ref[:] - m[None, :]) * inv[None, :] + b_ref[:]
  n0_ref[:] = y[:, :HH] + h0_ref[:]
  n1_ref[:] = y[:, HH:] + h1_ref[:]


def _bn_residual(u, part, g, b, h0, h1):
  return pl.pallas_call(
      _m2_body,
      grid=(NB_N,),
      in_specs=[
          pl.BlockSpec((BN_N, H), lambda i: (i, 0)),
          pl.BlockSpec((NB_N, 1, 2 * H), lambda i: (0, 0, 0)),
          pl.BlockSpec((1, H), lambda i: (0, 0)),
          pl.BlockSpec((1, H), lambda i: (0, 0)),
          pl.BlockSpec((BN_N, HH), lambda i: (i, 0)),
          pl.BlockSpec((BN_N, HH), lambda i: (i, 0)),
      ],
      out_specs=[
          pl.BlockSpec((BN_N, HH), lambda i: (i, 0)),
          pl.BlockSpec((BN_N, HH), lambda i: (i, 0)),
      ],
      out_shape=[
          jax.ShapeDtypeStruct((N, HH), jnp.float32),
          jax.ShapeDtypeStruct((N, HH), jnp.float32),
      ],
  )(u, part, g, b, h0, h1)


def _r_body(h0_ref, h1_ref, bid_ref, w_ref, b_ref, out_ref, pool_acc, cnt_acc):
  i = pl.program_id(0)

  @pl.when(i == 0)
  def _():
    pool_acc[:] = jnp.zeros_like(pool_acc)
    cnt_acc[:] = jnp.zeros_like(cnt_acc)

  iota = lax.broadcasted_iota(jnp.int32, (BN_N, NG), 1)
  a = _f32(bid_ref[:] == iota)
  hfull = jnp.concatenate([h0_ref[:], h1_ref[:]], axis=1)
  dn = (((0,), (0,)), ((), ()))
  pool_acc[:] = pool_acc[:] + lax.dot_general(
      a, hfull, dn, preferred_element_type=jnp.float32,
                  precision=lax.Precision.HIGHEST)
  cnt_acc[:] = cnt_acc[:] + lax.dot_general(
      a, jnp.ones((BN_N, NT), jnp.float32), dn,
      preferred_element_type=jnp.float32,
                  precision=lax.Precision.HIGHEST)

  @pl.when(i == NB_N - 1)
  def _():
    cnt = jnp.maximum(cnt_acc[:, 0:1], 1.0)
    pooled = pool_acc[:] / cnt
    out_ref[:] = jnp.dot(
        pooled.astype(jnp.bfloat16), w_ref[:].astype(jnp.bfloat16),
        preferred_element_type=jnp.float32) + b_ref[:]


def _readout(h0, h1, bid, fcl_W, fcl_b):
  return pl.pallas_call(
      _r_body,
      grid=(NB_N,),
      in_specs=[
          pl.BlockSpec((BN_N, HH), lambda i: (i, 0)),
          pl.BlockSpec((BN_N, HH), lambda i: (i, 0)),
          pl.BlockSpec((BN_N, 1), lambda i: (i, 0)),
          pl.BlockSpec((H, NT), lambda i: (0, 0)),
          pl.BlockSpec((1, NT), lambda i: (0, 0)),
      ],
      out_specs=pl.BlockSpec((NG, NT), lambda i: (0, 0)),
      out_shape=jax.ShapeDtypeStruct((NG, NT), jnp.float32),
      scratch_shapes=[
          pltpu.VMEM((NG, H), jnp.float32),
          pltpu.VMEM((NG, NT), jnp.float32),
      ],
  )(h0, h1, bid, fcl_W, fcl_b)


@jax.jit
def kernel(x, edge_index, edge_attr, batch_ids, W_atom, W_bond, eps, W_edge,
           W1, b1, W2, b2, bn_g, bn_b, bne_g, bne_b, bnin_g, bnin_b,
           fcl_W, fcl_b):
  x = x.astype(jnp.int32)
  ea = edge_attr.astype(jnp.int32)
  src = edge_index[0].astype(jnp.int32)
  dst = edge_index[1].astype(jnp.int32)
  bid = batch_ids.astype(jnp.int32).reshape(-1, 1)

  xe, pa = _atom_embed(x, W_atom)
  h0, h1 = _input_bn(xe, pa, bnin_g.reshape(1, -1), bnin_b.reshape(1, -1))
  zrows = jnp.zeros((NP, HH), jnp.float32)

  for i in range(L):
    e, pe = _bond_embed(ea, W_bond[i])
    ef0, ef1 = _edge_project(e, pe, bne_g[i].reshape(1, -1),
                             bne_b[i].reshape(1, -1), W_edge[i])
    a0, a1 = _sc_agg(src, dst, h0, h1, ef0, ef1, zrows)
    a0 = a0[:N]
    a1 = a1[:N]
    epsp = (1.0 + eps[i]).reshape(1, 1)
    u, pu = _mlp(h0, h1, a0, a1, epsp, W1[i], b1[i].reshape(1, -1),
                 W2[i], b2[i].reshape(1, -1))
    h0, h1 = _bn_residual(u, pu, bn_g[i].reshape(1, -1),
                          bn_b[i].reshape(1, -1), h0, h1)

  return _readout(h0, h1, bid, fcl_W, fcl_b.reshape(1, -1))


# speedup vs baseline: 4.3654x; 1.0865x over previous
import functools

import jax
import jax.numpy as jnp
from jax import lax
from jax.experimental import pallas as pl
from jax.experimental.pallas import tpu as pltpu, tpu_sc as plsc

N = 10000
E = 160000
H = 256
HH = 128
ED = 16
L = 3
NG = 256
NT = 128
AV = 64
BV = 16
NAF = 9
NBF = 3

NB_N = 10
BN_N = N // NB_N
NB_E = 80
BE = E // NB_E

SC_CORES = 2
SC_TILES = 16
LANES = 16
EPT = E // SC_TILES
CHUNK = 40
NP = 10240
RPT = NP // SC_TILES


def _f32(v):
  return v.astype(jnp.float32)


def _a1_body(x_ref, wa_ref, xe_ref, part_ref):
  acc = jnp.zeros((BN_N, H), jnp.float32)
  iota = lax.broadcasted_iota(jnp.int32, (BN_N, AV), 1)
  for f in range(NAF):
    oh = _f32(x_ref[:, f : f + 1] == iota)
    acc = acc + jnp.dot(oh, wa_ref[f], preferred_element_type=jnp.float32,
                  precision=lax.Precision.HIGHEST)
  xe_ref[:] = acc
  part_ref[0, :, :H] = jnp.sum(acc, axis=0, keepdims=True)
  part_ref[0, :, H:] = jnp.sum(acc * acc, axis=0, keepdims=True)


def _atom_embed(x, W_atom):
  return pl.pallas_call(
      _a1_body,
      grid=(NB_N,),
      in_specs=[
          pl.BlockSpec((BN_N, NAF), lambda i: (i, 0)),
          pl.BlockSpec((NAF, AV, H), lambda i: (0, 0, 0)),
      ],
      out_specs=[
          pl.BlockSpec((BN_N, H), lambda i: (i, 0)),
          pl.BlockSpec((1, 1, 2 * H), lambda i: (i, 0, 0)),
      ],
      out_shape=[
          jax.ShapeDtypeStruct((N, H), jnp.float32),
          jax.ShapeDtypeStruct((NB_N, 1, 2 * H), jnp.float32),
      ],
  )(x, W_atom)


def _bn_from_partials(part, n_rows, d):
  tot = jnp.sum(part, axis=(0, 1))
  m = tot[:d] * (1.0 / n_rows)
  ex2 = tot[d:] * (1.0 / n_rows)
  var = ex2 - m * m
  inv = lax.rsqrt(var + 1e-5)
  return m, inv


def _a2_body(xe_ref, part_ref, g_ref, b_ref, h0_ref, h1_ref):
  m, inv = _bn_from_partials(part_ref[:], N, H)
  y = g_ref[:] * (xe_ref[:] - m[None, :]) * inv[None, :] + b_ref[:]
  h0_ref[:] = y[:, :HH]
  h1_ref[:] = y[:, HH:]


def _input_bn(xe, part, g, b):
  return pl.pallas_call(
      _a2_body,
      grid=(NB_N,),
      in_specs=[
          pl.BlockSpec((BN_N, H), lambda i: (i, 0)),
          pl.BlockSpec((NB_N, 1, 2 * H), lambda i: (0, 0, 0)),
          pl.BlockSpec((1, H), lambda i: (0, 0)),
          pl.BlockSpec((1, H), lambda i: (0, 0)),
      ],
      out_specs=[
          pl.BlockSpec((BN_N, HH), lambda i: (i, 0)),
          pl.BlockSpec((BN_N, HH), lambda i: (i, 0)),
      ],
      out_shape=[
          jax.ShapeDtypeStruct((N, HH), jnp.float32),
          jax.ShapeDtypeStruct((N, HH), jnp.float32),
      ],
  )(xe, part, g, b)


def _e1_body(ea_ref, wb_ref, e_ref, part_ref):
  acc = jnp.zeros((BE, ED), jnp.float32)
  iota = lax.broadcasted_iota(jnp.int32, (BE, BV), 1)
  for f in range(NBF):
    oh = _f32(ea_ref[:, f : f + 1] == iota)
    acc = acc + jnp.dot(oh, wb_ref[f], preferred_element_type=jnp.float32,
                  precision=lax.Precision.HIGHEST)
  e_ref[:] = acc
  part_ref[0, :, :ED] = jnp.sum(acc, axis=0, keepdims=True)
  part_ref[0, :, ED:] = jnp.sum(acc * acc, axis=0, keepdims=True)


def _bond_embed(ea, Wb):
  return pl.pallas_call(
      _e1_body,
      grid=(NB_E,),
      in_specs=[
          pl.BlockSpec((BE, NBF), lambda i: (i, 0)),
          pl.BlockSpec((NBF, BV, ED), lambda i: (0, 0, 0)),
      ],
      out_specs=[
          pl.BlockSpec((BE, ED), lambda i: (i, 0)),
          pl.BlockSpec((1, 1, 2 * ED), lambda i: (i, 0, 0)),
      ],
      out_shape=[
          jax.ShapeDtypeStruct((E, ED), jnp.float32),
          jax.ShapeDtypeStruct((NB_E, 1, 2 * ED), jnp.float32),
      ],
  )(ea, Wb)


def _e3_body(e_ref, part_ref, g_ref, b_ref, we_ref, ef0_ref, ef1_ref):
  m, inv = _bn_from_partials(part_ref[:], E, ED)
  en = g_ref[:] * (e_ref[:] - m[None, :]) * inv[None, :] + b_ref[:]
  ef = jnp.dot(en.astype(jnp.bfloat16), we_ref[:].astype(jnp.bfloat16),
               preferred_element_type=jnp.float32)
  ef0_ref[:] = ef[:, :HH]
  ef1_ref[:] = ef[:, HH:]


def _edge_project(e, part, g, b, We):
  return pl.pallas_call(
      _e3_body,
      grid=(NB_E,),
      in_specs=[
          pl.BlockSpec((BE, ED), lambda i: (i, 0)),
          pl.BlockSpec((NB_E, 1, 2 * ED), lambda i: (0, 0, 0)),
          pl.BlockSpec((1, ED), lambda i: (0, 0)),
          pl.BlockSpec((1, ED), lambda i: (0, 0)),
          pl.BlockSpec((ED, H), lambda i: (0, 0)),
      ],
      out_specs=[
          pl.BlockSpec((BE, HH), lambda i: (i, 0)),
          pl.BlockSpec((BE, HH), lambda i: (i, 0)),
      ],
      out_shape=[
          jax.ShapeDtypeStruct((E, HH), jnp.float32),
          jax.ShapeDtypeStruct((E, HH), jnp.float32),
      ],
  )(e, part, g, b, We)


NBUF = 4
NCH = EPT // CHUNK
MAIN = (NCH // NBUF) * NBUF
PEEL = NCH - MAIN


def _sc_body(src_r, dst_r, h0_r, h1_r, ef0_r, ef1_r, z_r, out0_r, out1_r,
             *scr):
  sidx = list(scr[0:NBUF])
  didx = list(scr[NBUF:2 * NBUF])
  hbuf = list(scr[2 * NBUF:3 * NBUF])
  efbuf = list(scr[3 * NBUF:4 * NBUF])
  acc = scr[4 * NBUF]
  gsem = list(scr[4 * NBUF + 1:5 * NBUF + 1])
  ssem = list(scr[5 * NBUF + 1:6 * NBUF + 1])
  isem = list(scr[6 * NBUF + 1:7 * NBUF + 1])
  c = lax.axis_index("c")
  s = lax.axis_index("s")

  def run(h_r, ef_r, out_r):
    row0 = s * RPT
    pltpu.sync_copy(z_r.at[pl.ds(row0, RPT)], acc.at[pl.ds(row0, RPT)])
    plsc.subcore_barrier()
    ebase = s * EPT

    def prep_idx(k, b):
      off = ebase + k * CHUNK
      pltpu.async_copy(src_r.at[pl.ds(off, CHUNK)], sidx[b], isem[b])
      pltpu.async_copy(dst_r.at[pl.ds(off, CHUNK)], didx[b], isem[b])

    def wait_idx(k, b):
      off = ebase + k * CHUNK
      pltpu.make_async_copy(src_r.at[pl.ds(off, CHUNK)], sidx[b],
                            isem[b]).wait()
      pltpu.make_async_copy(dst_r.at[pl.ds(off, CHUNK)], didx[b],
                            isem[b]).wait()

    def prep_data(k, b):
      off = ebase + k * CHUNK
      pltpu.async_copy(h_r.at[sidx[b]], hbuf[b], gsem[b])
      pltpu.async_copy(ef_r.at[pl.ds(off, CHUNK)], efbuf[b], gsem[b])

    def wait_scatter(b):
      pltpu.make_async_copy(hbuf[b], acc.at[didx[b]], ssem[b]).wait()

    def consume(k, b):
      off = ebase + k * CHUNK
      pltpu.make_async_copy(h_r.at[sidx[b]], hbuf[b], gsem[b]).wait()
      pltpu.make_async_copy(ef_r.at[pl.ds(off, CHUNK)], efbuf[b],
                            gsem[b]).wait()

      @plsc.parallel_loop(0, CHUNK, 1, unroll=4)
      def rowfn(j):
        for q in range(HH // LANES):
          sl = pl.ds(q * LANES, LANES)
          hbuf[b][j, sl] = jnp.maximum(hbuf[b][j, sl] + efbuf[b][j, sl],
                                       0.0)

      pltpu.async_copy(hbuf[b], acc.at[didx[b]], ssem[b], add=True)

    prep_idx(0, 0)
    prep_idx(1, 1)
    wait_idx(0, 0)
    prep_data(0, 0)

    @pl.loop(0, MAIN, step=NBUF)
    def chunkgrp(k0):
      for db in range(NBUF):
        k = k0 + db
        b2 = (db + 2) % NBUF
        bn = (db + 1) % NBUF

        @pl.when(k >= 2)
        def _():
          wait_scatter(b2)

        prep_idx(k + 2, b2)
        wait_idx(k + 1, bn)
        prep_data(k + 1, bn)
        consume(k, db)

    wait_idx(MAIN + 1, 1)
    prep_data(MAIN + 1, 1)
    for j in range(PEEL):
      consume(MAIN + j, j)

    for b in range(NBUF):
      wait_scatter(b)
    plsc.subcore_barrier()
    pltpu.sync_copy(acc.at[pl.ds(row0, RPT)], out_r.at[pl.ds(row0, RPT)])

  @pl.when(c == 0)
  def _():
    run(h0_r, ef0_r, out0_r)

  @pl.when(c == 1)
  def _():
    run(h1_r, ef1_r, out1_r)


def _sc_agg(src, dst, h0, h1, ef0, ef1, zrows):
  sc_call = functools.partial(
      pl.kernel,
      out_type=[
          jax.ShapeDtypeStruct((NP, HH), jnp.float32),
          jax.ShapeDtypeStruct((NP, HH), jnp.float32),
      ],
      mesh=plsc.VectorSubcoreMesh(core_axis_name="c", subcore_axis_name="s"),
      scratch_types=(
          [pltpu.VMEM((CHUNK,), jnp.int32) for _ in range(2 * NBUF)]
          + [pltpu.VMEM((CHUNK, HH), jnp.float32) for _ in range(2 * NBUF)]
          + [pltpu.VMEM_SHARED((NP, HH), jnp.float32)]
          + [pltpu.SemaphoreType.DMA for _ in range(3 * NBUF)]
      ),
  )
  return sc_call(_sc_body)(src, dst, h0, h1, ef0, ef1, zrows)


def _m1_body(h0_ref, h1_ref, a0_ref, a1_ref, ep_ref, w1_ref, b1_ref,
             w2_ref, b2_ref, u_ref, part_ref):
  ep = ep_ref[0, 0]
  z0 = (h0_ref[:] * ep + a0_ref[:]).astype(jnp.bfloat16)
  z1 = (h1_ref[:] * ep + a1_ref[:]).astype(jnp.bfloat16)
  w1b = w1_ref[:].astype(jnp.bfloat16)
  t = jnp.dot(z0, w1b[:HH, :], preferred_element_type=jnp.float32)
  t = t + jnp.dot(z1, w1b[HH:, :], preferred_element_type=jnp.float32)
  t = jnp.maximum(t + b1_ref[:], 0.0)
  u = jnp.dot(t.astype(jnp.bfloat16), w2_ref[:].astype(jnp.bfloat16),
              preferred_element_type=jnp.float32) + b2_ref[:]
  u = jnp.maximum(u, 0.0)
  u_ref[:] = u
  part_ref[0, :, :H] = jnp.sum(u, axis=0, keepdims=True)
  part_ref[0, :, H:] = jnp.sum(u * u, axis=0, keepdims=True)


def _mlp(h0, h1, a0, a1, epsp, W1, b1, W2, b2):
  return pl.pallas_call(
      _m1_body,
      grid=(NB_N,),
      in_specs=[
          pl.BlockSpec((BN_N, HH), lambda i: (i, 0)),
          pl.BlockSpec((BN_N, HH), lambda i: (i, 0)),
          pl.BlockSpec((BN_N, HH), lambda i: (i, 0)),
          pl.BlockSpec((BN_N, HH), lambda i: (i, 0)),
          pl.BlockSpec((1, 1), lambda i: (0, 0)),
          pl.BlockSpec((H, H), lambda i: (0, 0)),
          pl.BlockSpec((1, H), lambda i: (0, 0)),
          pl.BlockSpec((H, H), lambda i: (0, 0)),
          pl.BlockSpec((1, H), lambda i: (0, 0)),
      ],
      out_specs=[
          pl.BlockSpec((BN_N, H), lambda i: (i, 0)),
          pl.BlockSpec((1, 1, 2 * H), lambda i: (i, 0, 0)),
      ],
      out_shape=[
          jax.ShapeDtypeStruct((N, H), jnp.float32),
          jax.ShapeDtypeStruct((NB_N, 1, 2 * H), jnp.float32),
      ],
  )(h0, h1, a0, a1, epsp, W1, b1, W2, b2)


def _m2_body(u_ref, part_ref, g_ref, b_ref, h0_ref, h1_ref, n0_ref, n1_ref):
  m, inv = _bn_from_partials(part_ref[:], N, H)
  y = g_ref[:] * (u_ref[:] - m[None, :]) * inv[None, :] + b_ref[:]
  n0_ref[:] = y[:, :HH] + h0_ref[:]
  n1_ref[:] = y[:, HH:] + h1_ref[:]


def _bn_residual(u, part, g, b, h0, h1):
  return pl.pallas_call(
      _m2_body,
      grid=(NB_N,),
      in_specs=[
          pl.BlockSpec((BN_N, H), lambda i: (i, 0)),
          pl.BlockSpec((NB_N, 1, 2 * H), lambda i: (0, 0, 0)),
          pl.BlockSpec((1, H), lambda i: (0, 0)),
          pl.BlockSpec((1, H), lambda i: (0, 0)),
          pl.BlockSpec((BN_N, HH), lambda i: (i, 0)),
          pl.BlockSpec((BN_N, HH), lambda i: (i, 0)),
      ],
      out_specs=[
          pl.BlockSpec((BN_N, HH), lambda i: (i, 0)),
          pl.BlockSpec((BN_N, HH), lambda i: (i, 0)),
      ],
      out_shape=[
          jax.ShapeDtypeStruct((N, HH), jnp.float32),
          jax.ShapeDtypeStruct((N, HH), jnp.float32),
      ],
  )(u, part, g, b, h0, h1)


def _r_body(h0_ref, h1_ref, bid_ref, w_ref, b_ref, out_ref, pool_acc, cnt_acc):
  i = pl.program_id(0)

  @pl.when(i == 0)
  def _():
    pool_acc[:] = jnp.zeros_like(pool_acc)
    cnt_acc[:] = jnp.zeros_like(cnt_acc)

  iota = lax.broadcasted_iota(jnp.int32, (BN_N, NG), 1)
  a = _f32(bid_ref[:] == iota)
  hfull = jnp.concatenate([h0_ref[:], h1_ref[:]], axis=1)
  dn = (((0,), (0,)), ((), ()))
  pool_acc[:] = pool_acc[:] + lax.dot_general(
      a, hfull, dn, preferred_element_type=jnp.float32,
                  precision=lax.Precision.HIGHEST)
  cnt_acc[:] = cnt_acc[:] + lax.dot_general(
      a, jnp.ones((BN_N, NT), jnp.float32), dn,
      preferred_element_type=jnp.float32,
                  precision=lax.Precision.HIGHEST)

  @pl.when(i == NB_N - 1)
  def _():
    cnt = jnp.maximum(cnt_acc[:, 0:1], 1.0)
    pooled = pool_acc[:] / cnt
    out_ref[:] = jnp.dot(
        pooled.astype(jnp.bfloat16), w_ref[:].astype(jnp.bfloat16),
        preferred_element_type=jnp.float32) + b_ref[:]


def _readout(h0, h1, bid, fcl_W, fcl_b):
  return pl.pallas_call(
      _r_body,
      grid=(NB_N,),
      in_specs=[
          pl.BlockSpec((BN_N, HH), lambda i: (i, 0)),
          pl.BlockSpec((BN_N, HH), lambda i: (i, 0)),
          pl.BlockSpec((BN_N, 1), lambda i: (i, 0)),
          pl.BlockSpec((H, NT), lambda i: (0, 0)),
          pl.BlockSpec((1, NT), lambda i: (0, 0)),
      ],
      out_specs=pl.BlockSpec((NG, NT), lambda i: (0, 0)),
      out_shape=jax.ShapeDtypeStruct((NG, NT), jnp.float32),
      scratch_shapes=[
          pltpu.VMEM((NG, H), jnp.float32),
          pltpu.VMEM((NG, NT), jnp.float32),
      ],
  )(h0, h1, bid, fcl_W, fcl_b)


@jax.jit
def kernel(x, edge_index, edge_attr, batch_ids, W_atom, W_bond, eps, W_edge,
           W1, b1, W2, b2, bn_g, bn_b, bne_g, bne_b, bnin_g, bnin_b,
           fcl_W, fcl_b):
  x = x.astype(jnp.int32)
  ea = edge_attr.astype(jnp.int32)
  src = edge_index[0].astype(jnp.int32)
  dst = edge_index[1].astype(jnp.int32)
  bid = batch_ids.astype(jnp.int32).reshape(-1, 1)

  xe, pa = _atom_embed(x, W_atom)
  h0, h1 = _input_bn(xe, pa, bnin_g.reshape(1, -1), bnin_b.reshape(1, -1))
  zrows = jnp.zeros((NP, HH), jnp.float32)

  for i in range(L):
    e, pe = _bond_embed(ea, W_bond[i])
    ef0, ef1 = _edge_project(e, pe, bne_g[i].reshape(1, -1),
                             bne_b[i].reshape(1, -1), W_edge[i])
    a0, a1 = _sc_agg(src, dst, h0, h1, ef0, ef1, zrows)
    a0 = a0[:N]
    a1 = a1[:N]
    epsp = (1.0 + eps[i]).reshape(1, 1)
    u, pu = _mlp(h0, h1, a0, a1, epsp, W1[i], b1[i].reshape(1, -1),
                 W2[i], b2[i].reshape(1, -1))
    h0, h1 = _bn_residual(u, pu, bn_g[i].reshape(1, -1),
                          bn_b[i].reshape(1, -1), h0, h1)

  return _readout(h0, h1, bid, fcl_W, fcl_b.reshape(1, -1))


# speedup vs baseline: 4.3698x; 1.0010x over previous
import functools

import jax
import jax.numpy as jnp
from jax import lax
from jax.experimental import pallas as pl
from jax.experimental.pallas import tpu as pltpu, tpu_sc as plsc

N = 10000
E = 160000
H = 256
HH = 128
ED = 16
L = 3
NG = 256
NT = 128
AV = 64
BV = 16
NAF = 9
NBF = 3

NB_N = 10
BN_N = N // NB_N
NB_E = 80
BE = E // NB_E
CV = NBF * BV
CVP = 128
AVC = NAF * AV

SC_CORES = 2
SC_TILES = 16
LANES = 16
EPT = E // SC_TILES
CHUNK = 40
NP = 10240
RPT = NP // SC_TILES


def _f32(v):
  return v.astype(jnp.float32)


def _a1_body(x_ref, wa_ref, xe_ref, part_ref):
  acc = jnp.zeros((BN_N, H), jnp.float32)
  iota = lax.broadcasted_iota(jnp.int32, (BN_N, AV), 1)
  for f in range(NAF):
    oh = _f32(x_ref[:, f : f + 1] == iota)
    acc = acc + jnp.dot(oh, wa_ref[f], preferred_element_type=jnp.float32,
                  precision=lax.Precision.HIGHEST)
  xe_ref[:] = acc
  part_ref[0, :, :H] = jnp.sum(acc, axis=0, keepdims=True)
  part_ref[0, :, H:] = jnp.sum(acc * acc, axis=0, keepdims=True)


def _atom_embed(x, W_atom):
  return pl.pallas_call(
      _a1_body,
      grid=(NB_N,),
      in_specs=[
          pl.BlockSpec((BN_N, NAF), lambda i: (i, 0)),
          pl.BlockSpec((NAF, AV, H), lambda i: (0, 0, 0)),
      ],
      out_specs=[
          pl.BlockSpec((BN_N, H), lambda i: (i, 0)),
          pl.BlockSpec((1, 1, 2 * H), lambda i: (i, 0, 0)),
      ],
      out_shape=[
          jax.ShapeDtypeStruct((N, H), jnp.float32),
          jax.ShapeDtypeStruct((NB_N, 1, 2 * H), jnp.float32),
      ],
  )(x, W_atom)


def _bn_from_partials(part, n_rows, d):
  tot = jnp.sum(part, axis=(0, 1))
  m = tot[:d] * (1.0 / n_rows)
  ex2 = tot[d:] * (1.0 / n_rows)
  var = ex2 - m * m
  inv = lax.rsqrt(var + 1e-5)
  return m, inv


def _a2_body(xe_ref, part_ref, g_ref, b_ref, h0_ref, h1_ref):
  m, inv = _bn_from_partials(part_ref[:], N, H)
  y = g_ref[:] * (xe_ref[:] - m[None, :]) * inv[None, :] + b_ref[:]
  h0_ref[:] = y[:, :HH]
  h1_ref[:] = y[:, HH:]


def _input_bn(xe, part, g, b):
  return pl.pallas_call(
      _a2_body,
      grid=(NB_N,),
      in_specs=[
          pl.BlockSpec((BN_N, H), lambda i: (i, 0)),
          pl.BlockSpec((NB_N, 1, 2 * H), lambda i: (0, 0, 0)),
          pl.BlockSpec((1, H), lambda i: (0, 0)),
          pl.BlockSpec((1, H), lambda i: (0, 0)),
      ],
      out_specs=[
          pl.BlockSpec((BN_N, HH), lambda i: (i, 0)),
          pl.BlockSpec((BN_N, HH), lambda i: (i, 0)),
      ],
      out_shape=[
          jax.ShapeDtypeStruct((N, HH), jnp.float32),
          jax.ShapeDtypeStruct((N, HH), jnp.float32),
      ],
  )(xe, part, g, b)


def _e1_body(ea_ref, wb_ref, e_ref, part_ref):
  acc = jnp.zeros((BE, ED), jnp.float32)
  iota = lax.broadcasted_iota(jnp.int32, (BE, BV), 1)
  for f in range(NBF):
    oh = _f32(ea_ref[:, f : f + 1] == iota)
    acc = acc + jnp.dot(oh, wb_ref[f], preferred_element_type=jnp.float32,
                  precision=lax.Precision.HIGHEST)
  e_ref[:] = acc
  part_ref[0, :, :ED] = jnp.sum(acc, axis=0, keepdims=True)
  part_ref[0, :, ED:] = jnp.sum(acc * acc, axis=0, keepdims=True)


def _bond_embed(ea, Wb):
  return pl.pallas_call(
      _e1_body,
      grid=(NB_E,),
      in_specs=[
          pl.BlockSpec((BE, NBF), lambda i: (i, 0)),
          pl.BlockSpec((NBF, BV, ED), lambda i: (0, 0, 0)),
      ],
      out_specs=[
          pl.BlockSpec((BE, ED), lambda i: (i, 0)),
          pl.BlockSpec((1, 1, 2 * ED), lambda i: (i, 0, 0)),
      ],
      out_shape=[
          jax.ShapeDtypeStruct((E, ED), jnp.float32),
          jax.ShapeDtypeStruct((NB_E, 1, 2 * ED), jnp.float32),
      ],
  )(ea, Wb)


def _e3_body(e_ref, part_ref, g_ref, b_ref, we_ref, ef0_ref, ef1_ref):
  m, inv = _bn_from_partials(part_ref[:], E, ED)
  en = g_ref[:] * (e_ref[:] - m[None, :]) * inv[None, :] + b_ref[:]
  ef = jnp.dot(en.astype(jnp.bfloat16), we_ref[:].astype(jnp.bfloat16),
               preferred_element_type=jnp.float32)
  ef0_ref[:] = ef[:, :HH]
  ef1_ref[:] = ef[:, HH:]


def _edge_project(e, part, g, b, We):
  return pl.pallas_call(
      _e3_body,
      grid=(NB_E,),
      in_specs=[
          pl.BlockSpec((BE, ED), lambda i: (i, 0)),
          pl.BlockSpec((NB_E, 1, 2 * ED), lambda i: (0, 0, 0)),
          pl.BlockSpec((1, ED), lambda i: (0, 0)),
          pl.BlockSpec((1, ED), lambda i: (0, 0)),
          pl.BlockSpec((ED, H), lambda i: (0, 0)),
      ],
      out_specs=[
          pl.BlockSpec((BE, HH), lambda i: (i, 0)),
          pl.BlockSpec((BE, HH), lambda i: (i, 0)),
      ],
      out_shape=[
          jax.ShapeDtypeStruct((E, HH), jnp.float32),
          jax.ShapeDtypeStruct((E, HH), jnp.float32),
      ],
  )(e, part, g, b, We)


NBUF = 4
NCH = EPT // CHUNK
MAIN = (NCH // NBUF) * NBUF
PEEL = NCH - MAIN


def _sc_body(src_r, dst_r, h0_r, h1_r, ef0_r, ef1_r, z_r, out0_r, out1_r,
             *scr):
  sidx = list(scr[0:NBUF])
  didx = list(scr[NBUF:2 * NBUF])
  hbuf = list(scr[2 * NBUF:3 * NBUF])
  efbuf = list(scr[3 * NBUF:4 * NBUF])
  acc = scr[4 * NBUF]
  gsem = list(scr[4 * NBUF + 1:5 * NBUF + 1])
  ssem = list(scr[5 * NBUF + 1:6 * NBUF + 1])
  isem = list(scr[6 * NBUF + 1:7 * NBUF + 1])
  c = lax.axis_index("c")
  s = lax.axis_index("s")

  def run(h_r, ef_r, out_r):
    row0 = s * RPT
    pltpu.sync_copy(z_r.at[pl.ds(row0, RPT)], acc.at[pl.ds(row0, RPT)])
    plsc.subcore_barrier()
    ebase = s * EPT

    def prep_idx(k, b):
      off = ebase + k * CHUNK
      pltpu.async_copy(src_r.at[pl.ds(off, CHUNK)], sidx[b], isem[b])
      pltpu.async_copy(dst_r.at[pl.ds(off, CHUNK)], didx[b], isem[b])

    def wait_idx(k, b):
      off = ebase + k * CHUNK
      pltpu.make_async_copy(src_r.at[pl.ds(off, CHUNK)], sidx[b],
                            isem[b]).wait()
      pltpu.make_async_copy(dst_r.at[pl.ds(off, CHUNK)], didx[b],
                            isem[b]).wait()

    def prep_data(k, b):
      off = ebase + k * CHUNK
      pltpu.async_copy(h_r.at[sidx[b]], hbuf[b], gsem[b])
      pltpu.async_copy(ef_r.at[pl.ds(off, CHUNK)], efbuf[b], gsem[b])

    def wait_scatter(b):
      pltpu.make_async_copy(hbuf[b], acc.at[didx[b]], ssem[b]).wait()

    def consume(k, b):
      off = ebase + k * CHUNK
      pltpu.make_async_copy(h_r.at[sidx[b]], hbuf[b], gsem[b]).wait()
      pltpu.make_async_copy(ef_r.at[pl.ds(off, CHUNK)], efbuf[b],
                            gsem[b]).wait()

      @plsc.parallel_loop(0, CHUNK, 1, unroll=4)
      def rowfn(j):
        for q in range(HH // LANES):
          sl = pl.ds(q * LANES, LANES)
          hbuf[b][j, sl] = jnp.maximum(hbuf[b][j, sl] + efbuf[b][j, sl],
                                       0.0)

      pltpu.async_copy(hbuf[b], acc.at[didx[b]], ssem[b], add=True)

    prep_idx(0, 0)
    prep_idx(1, 1)
    wait_idx(0, 0)
    prep_data(0, 0)

    @pl.loop(0, MAIN, step=NBUF)
    def chunkgrp(k0):
      for db in range(NBUF):
        k = k0 + db
        b2 = (db + 2) % NBUF
        bn = (db + 1) % NBUF

        @pl.when(k >= 2)
        def _():
          wait_scatter(b2)

        prep_idx(k + 2, b2)
        wait_idx(k + 1, bn)
        prep_data(k + 1, bn)
        consume(k, db)

    wait_idx(MAIN + 1, 1)
    prep_data(MAIN + 1, 1)
    for j in range(PEEL):
      consume(MAIN + j, j)

    for b in range(NBUF):
      wait_scatter(b)
    plsc.subcore_barrier()
    pltpu.sync_copy(acc.at[pl.ds(row0, RPT)], out_r.at[pl.ds(row0, RPT)])

  @pl.when(c == 0)
  def _():
    run(h0_r, ef0_r, out0_r)

  @pl.when(c == 1)
  def _():
    run(h1_r, ef1_r, out1_r)


def _sc_agg(src, dst, h0, h1, ef0, ef1, zrows):
  sc_call = functools.partial(
      pl.kernel,
      out_type=[
          jax.ShapeDtypeStruct((NP, HH), jnp.float32),
          jax.ShapeDtypeStruct((NP, HH), jnp.float32),
      ],
      mesh=plsc.VectorSubcoreMesh(core_axis_name="c", subcore_axis_name="s"),
      scratch_types=(
          [pltpu.VMEM((CHUNK,), jnp.int32) for _ in range(2 * NBUF)]
          + [pltpu.VMEM((CHUNK, HH), jnp.float32) for _ in range(2 * NBUF)]
          + [pltpu.VMEM_SHARED((NP, HH), jnp.float32)]
          + [pltpu.SemaphoreType.DMA for _ in range(3 * NBUF)]
      ),
  )
  return sc_call(_sc_body)(src, dst, h0, h1, ef0, ef1, zrows)


def _m1_body(h0_ref, h1_ref, a0_ref, a1_ref, ep_ref, w1_ref, b1_ref,
             w2_ref, b2_ref, u_ref, part_ref):
  ep = ep_ref[0, 0]
  z0 = (h0_ref[:] * ep + a0_ref[:]).astype(jnp.bfloat16)
  z1 = (h1_ref[:] * ep + a1_ref[:]).astype(jnp.bfloat16)
  w1b = w1_ref[:].astype(jnp.bfloat16)
  t = jnp.dot(z0, w1b[:HH, :], preferred_element_type=jnp.float32)
  t = t + jnp.dot(z1, w1b[HH:, :], preferred_element_type=jnp.float32)
  t = jnp.maximum(t + b1_ref[:], 0.0)
  u = jnp.dot(t.astype(jnp.bfloat16), w2_ref[:].astype(jnp.bfloat16),
              preferred_element_type=jnp.float32) + b2_ref[:]
  u = jnp.maximum(u, 0.0)
  u_ref[:] = u
  part_ref[0, :, :H] = jnp.sum(u, axis=0, keepdims=True)
  part_ref[0, :, H:] = jnp.sum(u * u, axis=0, keepdims=True)


def _mlp(h0, h1, a0, a1, epsp, W1, b1, W2, b2):
  return pl.pallas_call(
      _m1_body,
      grid=(NB_N,),
      in_specs=[
          pl.BlockSpec((BN_N, HH), lambda i: (i, 0)),
          pl.BlockSpec((BN_N, HH), lambda i: (i, 0)),
          pl.BlockSpec((BN_N, HH), lambda i: (i, 0)),
          pl.BlockSpec((BN_N, HH), lambda i: (i, 0)),
          pl.BlockSpec((1, 1), lambda i: (0, 0)),
          pl.BlockSpec((H, H), lambda i: (0, 0)),
          pl.BlockSpec((1, H), lambda i: (0, 0)),
          pl.BlockSpec((H, H), lambda i: (0, 0)),
          pl.BlockSpec((1, H), lambda i: (0, 0)),
      ],
      out_specs=[
          pl.BlockSpec((BN_N, H), lambda i: (i, 0)),
          pl.BlockSpec((1, 1, 2 * H), lambda i: (i, 0, 0)),
      ],
      out_shape=[
          jax.ShapeDtypeStruct((N, H), jnp.float32),
          jax.ShapeDtypeStruct((NB_N, 1, 2 * H), jnp.float32),
      ],
  )(h0, h1, a0, a1, epsp, W1, b1, W2, b2)


def _m2_body(u_ref, part_ref, g_ref, b_ref, h0_ref, h1_ref, n0_ref, n1_ref):
  m, inv = _bn_from_partials(part_ref[:], N, H)
  y = g_ref[:] * (u_ref[:] - m[None, :]) * inv[None, :] + b_ref[:]
  n0_ref[:] = y[:, :HH] + h0_ref[:]
  n1_ref[:] = y[:, HH:] + h1_ref[:]


def _bn_residual(u, part, g, b, h0, h1):
  return pl.pallas_call(
      _m2_body,
      grid=(NB_N,),
      in_specs=[
          pl.BlockSpec((BN_N, H), lambda i: (i, 0)),
          pl.BlockSpec((NB_N, 1, 2 * H), lambda i: (0, 0, 0)),
          pl.BlockSpec((1, H), lambda i: (0, 0)),
          pl.BlockSpec((1, H), lambda i: (0, 0)),
          pl.BlockSpec((BN_N, HH), lambda i: (i, 0)),
          pl.BlockSpec((BN_N, HH), lambda i: (i, 0)),
      ],
      out_specs=[
          pl.BlockSpec((BN_N, HH), lambda i: (i, 0)),
          pl.BlockSpec((BN_N, HH), lambda i: (i, 0)),
      ],
      out_shape=[
          jax.ShapeDtypeStruct((N, HH), jnp.float32),
          jax.ShapeDtypeStruct((N, HH), jnp.float32),
      ],
  )(u, part, g, b, h0, h1)


def _r_body(h0_ref, h1_ref, bid_ref, w_ref, b_ref, out_ref, pool_acc, cnt_acc):
  i = pl.program_id(0)

  @pl.when(i == 0)
  def _():
    pool_acc[:] = jnp.zeros_like(pool_acc)
    cnt_acc[:] = jnp.zeros_like(cnt_acc)

  iota = lax.broadcasted_iota(jnp.int32, (BN_N, NG), 1)
  a = _f32(bid_ref[:] == iota)
  hfull = jnp.concatenate([h0_ref[:], h1_ref[:]], axis=1)
  dn = (((0,), (0,)), ((), ()))
  pool_acc[:] = pool_acc[:] + lax.dot_general(
      a, hfull, dn, preferred_element_type=jnp.float32,
                  precision=lax.Precision.HIGHEST)
  cnt_acc[:] = cnt_acc[:] + lax.dot_general(
      a, jnp.ones((BN_N, NT), jnp.float32), dn,
      preferred_element_type=jnp.float32,
                  precision=lax.Precision.HIGHEST)

  @pl.when(i == NB_N - 1)
  def _():
    cnt = jnp.maximum(cnt_acc[:, 0:1], 1.0)
    pooled = pool_acc[:] / cnt
    out_ref[:] = jnp.dot(
        pooled.astype(jnp.bfloat16), w_ref[:].astype(jnp.bfloat16),
        preferred_element_type=jnp.float32) + b_ref[:]


def _readout(h0, h1, bid, fcl_W, fcl_b):
  return pl.pallas_call(
      _r_body,
      grid=(NB_N,),
      in_specs=[
          pl.BlockSpec((BN_N, HH), lambda i: (i, 0)),
          pl.BlockSpec((BN_N, HH), lambda i: (i, 0)),
          pl.BlockSpec((BN_N, 1), lambda i: (i, 0)),
          pl.BlockSpec((H, NT), lambda i: (0, 0)),
          pl.BlockSpec((1, NT), lambda i: (0, 0)),
      ],
      out_specs=pl.BlockSpec((NG, NT), lambda i: (0, 0)),
      out_shape=jax.ShapeDtypeStruct((NG, NT), jnp.float32),
      scratch_shapes=[
          pltpu.VMEM((NG, H), jnp.float32),
          pltpu.VMEM((NG, NT), jnp.float32),
      ],
  )(h0, h1, bid, fcl_W, fcl_b)


@jax.jit
def kernel(x, edge_index, edge_attr, batch_ids, W_atom, W_bond, eps, W_edge,
           W1, b1, W2, b2, bn_g, bn_b, bne_g, bne_b, bnin_g, bnin_b,
           fcl_W, fcl_b):
  x = x.astype(jnp.int32)
  ea = edge_attr.astype(jnp.int32)
  src = edge_index[0].astype(jnp.int32)
  dst = edge_index[1].astype(jnp.int32)
  bid = batch_ids.astype(jnp.int32).reshape(-1, 1)

  xe, pa = _atom_embed(x, W_atom)
  h0, h1 = _input_bn(xe, pa, bnin_g.reshape(1, -1), bnin_b.reshape(1, -1))
  zrows = jnp.zeros((NP, HH), jnp.float32)

  efs = []
  for i in range(L):
    e, pe = _bond_embed(ea, W_bond[i])
    ef0, ef1 = _edge_project(e, pe, bne_g[i].reshape(1, -1),
                             bne_b[i].reshape(1, -1), W_edge[i])
    efs.extend([ef0, ef1])

  for i in range(L):
    ef0, ef1 = efs[2 * i], efs[2 * i + 1]
    a0, a1 = _sc_agg(src, dst, h0, h1, ef0, ef1, zrows)
    a0 = a0[:N]
    a1 = a1[:N]
    epsp = (1.0 + eps[i]).reshape(1, 1)
    u, pu = _mlp(h0, h1, a0, a1, epsp, W1[i], b1[i].reshape(1, -1),
                 W2[i], b2[i].reshape(1, -1))
    h0, h1 = _bn_residual(u, pu, bn_g[i].reshape(1, -1),
                          bn_b[i].reshape(1, -1), h0, h1)

  return _readout(h0, h1, bid, fcl_W, fcl_b.reshape(1, -1))
